# Initial kernel scaffold; baseline (speedup 1.0000x reference)
import jax
import jax.numpy as jnp
from jax.experimental import pallas as pl


def kernel(x, table, W1, b1, W2, b2, W3, b3):
    raise NotImplementedError("write your pallas kernel here")


# speedup vs baseline: 27.3818x; 27.3818x over previous
import functools

import numpy as np
import jax
import jax.numpy as jnp
from jax import lax
from jax.experimental import pallas as pl
from jax.experimental.pallas import tpu as pltpu
from jax.experimental.pallas import tpu_sc as plsc

_L = 16
_F = 2
_BASE = 16
_T = 2 ** 19
_BOUND = 11.0
_MLPW = 64
_P1 = int(np.uint32(2654435761).astype(np.int32))
_P2 = 805459861

_NC, _NS, _LANES = 2, 16, 16
_NW = _NC * _NS

_C = 80
_G = _C // _LANES
_R = 8 * _C


def _build_encode(N):
    NCH = N // _C
    TI = (NCH + _NW - 1) // _NW

    mesh = plsc.VectorSubcoreMesh(core_axis_name="c", subcore_axis_name="s",
                                  num_cores=_NC, num_subcores=_NS)

    def body(xt_hbm, tab_hbm, out_hbm, xt_v, idx_v, gat_v, feat_v, sem):
        wid = lax.axis_index("s") * _NC + lax.axis_index("c")
        iot = lax.broadcasted_iota(jnp.int32, (_LANES,), 0)
        zer = jnp.zeros((_LANES,), jnp.int32)
        one = jnp.ones((_LANES,), jnp.int32)

        def chunk_body(t, carry):
            ch = wid + t * _NW

            @pl.when(ch < NCH)
            def _():
                base = ch * _C
                for d in range(3):
                    pltpu.sync_copy(xt_hbm.at[pl.ds(d * N + base, _C)], xt_v.at[pl.ds(d * _C, _C)])

                def lvl_idx(l, c0_):
                    res = lax.shift_left(jnp.int32(_BASE), l)
                    res1 = res + 1
                    res1sq = res1 * res1
                    sf = (res - 1).astype(jnp.float32)
                    dense = lax.broadcast(l < 3, (_LANES,))
                    lbase = l * _T

                    def grp(g, c1_):
                        off = g * _LANES
                        c = []
                        for d in range(3):
                            xv = xt_v[pl.ds(d * _C + off, _LANES)]
                            xn = jnp.clip((xv + _BOUND) * (0.5 / _BOUND), 0.0, 1.0)
                            pos = xn * sf + 0.5
                            c.append(pos.astype(jnp.int32))
                        m1d = (c[1] * res1, c[1] * res1 + res1)
                        m2d = (c[2] * res1sq, c[2] * res1sq + res1sq)
                        m1h = (c[1] * _P1, c[1] * _P1 + _P1)
                        m2h = (c[2] * _P2, c[2] * _P2 + _P2)
                        for i in (0, 1):
                            a = c[0] + i
                            for j in (0, 1):
                                for k in (0, 1):
                                    n = i * 4 + j * 2 + k
                                    dsum = a + m1d[j] + m2d[k]
                                    hsum = (a ^ m1h[j] ^ m2h[k]) & (_T - 1)
                                    idx = jnp.where(dense, dsum, hsum) + lbase
                                    idx_v[pl.ds(l * _R + n * _C + off, _LANES)] = idx
                        return c1_
                    lax.fori_loop(0, _G, grp, 0)
                    pltpu.async_copy(tab_hbm.at[idx_v.at[pl.ds(l * _R, _R)]],
                                     gat_v.at[pl.ds(l * _R, _R)], sem)
                    return c0_
                lax.fori_loop(0, _L, lvl_idx, 0)

                pltpu.make_async_copy(tab_hbm.at[idx_v], gat_v, sem).wait()

                def lvl_acc(l, c0_):
                    res = lax.shift_left(jnp.int32(_BASE), l)
                    sf = (res - 1).astype(jnp.float32)

                    def grp(g, c1_):
                        off = g * _LANES
                        fr = []
                        for d in range(3):
                            xv = xt_v[pl.ds(d * _C + off, _LANES)]
                            xn = jnp.clip((xv + _BOUND) * (0.5 / _BOUND), 0.0, 1.0)
                            pos = xn * sf + 0.5
                            pf = pos.astype(jnp.int32).astype(jnp.float32)
                            fr.append(pos - pf)
                        wx = (1.0 - fr[0], fr[0])
                        wy = (1.0 - fr[1], fr[1])
                        wz = (1.0 - fr[2], fr[2])
                        wyz = [wy[j] * wz[k] for j in (0, 1) for k in (0, 1)]
                        acc0 = jnp.zeros((_LANES,), jnp.float32)
                        acc1 = jnp.zeros((_LANES,), jnp.float32)
                        for i in (0, 1):
                            for j in (0, 1):
                                for k in (0, 1):
                                    n = i * 4 + j * 2 + k
                                    w = wx[i] * wyz[j * 2 + k]
                                    ridx = l * _R + n * _C + off + iot
                                    g0 = plsc.load_gather(gat_v, [ridx, zer])
                                    g1 = plsc.load_gather(gat_v, [ridx, one])
                                    acc0 = acc0 + w * g0
                                    acc1 = acc1 + w * g1
                        sidx = (off + iot) * (_L * _F) + lax.broadcast(2 * l, (_LANES,))
                        plsc.store_scatter(feat_v, [sidx], acc0)
                        plsc.store_scatter(feat_v, [sidx + one], acc1)
                        return c1_
                    lax.fori_loop(0, _G, grp, 0)
                    return c0_
                lax.fori_loop(0, _L, lvl_acc, 0)

                pltpu.sync_copy(feat_v, out_hbm.at[pl.ds(base * (_L * _F), _C * _L * _F)])
            return carry
        lax.fori_loop(0, TI, chunk_body, 0)

    return pl.kernel(
        body,
        out_type=jax.ShapeDtypeStruct((N * _L * _F,), jnp.float32),
        mesh=mesh,
        compiler_params=pltpu.CompilerParams(use_tc_tiling_on_sc=False,
                                             needs_layout_passes=False),
        scratch_types=[
            pltpu.VMEM((3 * _C,), jnp.float32),
            pltpu.VMEM((_L * _R,), jnp.int32),
            pltpu.VMEM((_L * _R, _F), jnp.float32),
            pltpu.VMEM((_C * _L * _F,), jnp.float32),
            pltpu.SemaphoreType.DMA,
        ],
    )


@functools.lru_cache(maxsize=None)
def _get_encode(N):
    return _build_encode(N)


def _mlp(feats, W1, b1, W2, b2, W3, b3):
    N = feats.shape[0]
    BN = 8000
    assert N % BN == 0

    def mlp_body(f_ref, w1, b1r, w2, b2r, w3, b3r, o_ref):
        h = jnp.dot(f_ref[...], w1[...], preferred_element_type=jnp.float32)
        h = jnp.maximum(h + b1r[...], 0.0)
        h = jnp.dot(h, w2[...], preferred_element_type=jnp.float32)
        h = jnp.maximum(h + b2r[...], 0.0)
        o_ref[...] = jnp.dot(h, w3[...], preferred_element_type=jnp.float32) + b3r[...]

    return pl.pallas_call(
        mlp_body,
        grid=(N // BN,),
        in_specs=[
            pl.BlockSpec((BN, _L * _F), lambda i: (i, 0)),
            pl.BlockSpec((_L * _F, _MLPW), lambda i: (0, 0)),
            pl.BlockSpec((1, _MLPW), lambda i: (0, 0)),
            pl.BlockSpec((_MLPW, _MLPW), lambda i: (0, 0)),
            pl.BlockSpec((1, _MLPW), lambda i: (0, 0)),
            pl.BlockSpec((_MLPW, 1), lambda i: (0, 0)),
            pl.BlockSpec((1, 1), lambda i: (0, 0)),
        ],
        out_specs=pl.BlockSpec((BN, 1), lambda i: (i, 0)),
        out_shape=jax.ShapeDtypeStruct((N, 1), jnp.float32),
    )(feats, W1, b1.reshape(1, _MLPW), W2, b2.reshape(1, _MLPW),
      W3, b3.reshape(1, 1))


def kernel(x, table, W1, b1, W2, b2, W3, b3):
    N = x.shape[0]
    xt = x.T.reshape(3 * N)
    tab2 = table.reshape(_L * _T, _F)
    feats = _get_encode(N)(xt, tab2).reshape(N, _L * _F)
    return _mlp(feats, W1, b1, W2, b2, W3, b3)
